# Initial kernel scaffold; baseline (speedup 1.0000x reference)
import jax
import jax.numpy as jnp
from jax.experimental import pallas as pl


def kernel(n_feat, edge_index, edge_type, ori_edge_ids, rel_table, W0, b0, bn_scale, bn_bias, bn_mean, bn_var, W1, b1):
    raise NotImplementedError("write your pallas kernel here")


# speedup vs baseline: 6.2477x; 6.2477x over previous
import functools
import jax
import jax.numpy as jnp
from jax import lax
from jax.experimental import pallas as pl
from jax.experimental.pallas import tpu as pltpu
from jax.experimental.pallas import tpu_sc as plsc

N = 10000
E = 320000
E_ORI = 160000
D = 128
R_DIM = 16
N_REL = 16
HID = 64
LAMDA = 0.5
THRESH = 0.01

NC = 2
NS = 16
NW = NC * NS
L = 16

R = E // 128
RO = E_ORI // 128
ROWS_A = -(-R // NW)
ROWS_C = -(-R // NS)
ROWS_O = -(-RO // NS)

_mesh = plsc.VectorSubcoreMesh(
    core_axis_name="c", subcore_axis_name="s", num_cores=NC, num_subcores=NS)


def _ka_body(n_feat, row2d, col2d, et2d, rel_flat,
             x1, x2f,
             ridx, cidx, eidx, src, dst, rel_vmem, relrows, sem):
    c = lax.axis_index("c")
    s = lax.axis_index("s")
    wid = s * NC + c
    iota16 = lax.iota(jnp.int32, L)

    pltpu.sync_copy(rel_flat, rel_vmem)

    def body(t, carry):
        r = wid + t * NW

        @pl.when(r < R)
        def _():
            pltpu.sync_copy(row2d.at[r], ridx)
            pltpu.sync_copy(col2d.at[r], cidx)
            pltpu.sync_copy(et2d.at[r], eidx)
            pltpu.async_copy(n_feat.at[ridx], src, sem).wait()
            pltpu.async_copy(n_feat.at[cidx], dst, sem).wait()

            def row_i(i, carry2):
                for j in range(D // L):
                    sl = pl.ds(j * L, L)
                    src[i, sl] = -jnp.abs(src[i, sl] - dst[i, sl])
                return carry2
            lax.fori_loop(0, 128, row_i, 0)

            def rel_m(m, carry2):
                ev = eidx[pl.ds(m * L, L)]
                base = ev * R_DIM
                dst_base = m * (L * R_DIM) + iota16 * R_DIM
                for j in range(R_DIM):
                    vals = plsc.load_gather(rel_vmem, [base + j])
                    plsc.store_scatter(relrows, [dst_base + j], vals)
                return carry2
            lax.fori_loop(0, 128 // L, rel_m, 0)

            pltpu.sync_copy(src, x1.at[pl.ds(r * 128, 128), :])
            pltpu.sync_copy(relrows, x2f.at[pl.ds(r * 128 * R_DIM,
                                                  128 * R_DIM)])
        return carry

    lax.fori_loop(0, ROWS_A, body, 0)


_ka = pl.kernel(
    _ka_body,
    out_type=(
        jax.ShapeDtypeStruct((E, D), jnp.float32),
        jax.ShapeDtypeStruct((E * R_DIM,), jnp.float32),
    ),
    mesh=_mesh,
    scratch_types=(
        pltpu.VMEM((128,), jnp.int32),
        pltpu.VMEM((128,), jnp.int32),
        pltpu.VMEM((128,), jnp.int32),
        pltpu.VMEM((128, D), jnp.float32),
        pltpu.VMEM((128, D), jnp.float32),
        pltpu.VMEM((N_REL * R_DIM,), jnp.float32),
        pltpu.VMEM((128 * R_DIM,), jnp.float32),
        pltpu.SemaphoreType.DMA,
    ),
    compiler_params=pltpu.CompilerParams(needs_layout_passes=False),
)


BB = 2560


def _kb_body(x1_ref, x2_ref, w0a_ref, w0b_ref, bias_ref, w1_ref, b1_ref,
             out_ref):
    e1 = jnp.exp(x1_ref[...])
    h = (jnp.dot(e1, w0a_ref[...], preferred_element_type=jnp.float32)
         + jnp.dot(x2_ref[...], w0b_ref[...],
                   preferred_element_type=jnp.float32)
         + bias_ref[...])
    h = jnp.where(h >= 0.0, h, 0.01 * h)
    w = jnp.dot(h, w1_ref[...], preferred_element_type=jnp.float32)
    out_ref[...] = w + b1_ref[0, 0]


_kb = pl.pallas_call(
    _kb_body,
    out_shape=jax.ShapeDtypeStruct((E, 1), jnp.float32),
    grid=(E // BB,),
    in_specs=[
        pl.BlockSpec((BB, D), lambda i: (i, 0)),
        pl.BlockSpec((BB, R_DIM), lambda i: (i, 0)),
        pl.BlockSpec((D, HID), lambda i: (0, 0)),
        pl.BlockSpec((R_DIM, HID), lambda i: (0, 0)),
        pl.BlockSpec((1, HID), lambda i: (0, 0)),
        pl.BlockSpec((HID, 1), lambda i: (0, 0)),
        pl.BlockSpec((1, 1), lambda i: (0, 0)),
    ],
    out_specs=pl.BlockSpec((BB, 1), lambda i: (i, 0)),
)


def _kc_body(w_hbm, col2d, ori2d,
             outw,
             flag_sh, seg_sh, v_sh,
             zb, ones, oix, colb, wv, fv, vv, sv, ov, segv):
    c = lax.axis_index("c")
    s = lax.axis_index("s")
    tid = s
    wid = s * NC + c
    zero16 = jnp.zeros((L,), jnp.float32)
    one16 = jnp.ones((L,), jnp.float32)

    def z_i(i, carry):
        zb[pl.ds(i * L, L)] = zero16
        return carry
    lax.fori_loop(0, 2048 // L, z_i, 0)

    def o_i(i, carry):
        ones[pl.ds(i * L, L)] = one16
        return carry
    lax.fori_loop(0, 128 // L, o_i, 0)

    span = E // NS

    def zf_k(k, carry):
        pltpu.sync_copy(zb, flag_sh.at[pl.ds(tid * span + k * 2048, 2048)])
        return carry
    lax.fori_loop(0, 9, zf_k, 0)
    pltpu.sync_copy(zb.at[pl.ds(0, 1568)],
                    flag_sh.at[pl.ds(tid * span + 9 * 2048, 1568)])

    @pl.when(tid < 10)
    def _():
        pltpu.sync_copy(zb.at[pl.ds(0, 1000)],
                        seg_sh.at[pl.ds(tid * 1000, 1000)])

    plsc.subcore_barrier()

    def ph1(k, carry):
        j = tid + k * NS

        @pl.when(j < RO)
        def _():
            pltpu.sync_copy(ori2d.at[pl.ds(j, 1), :], oix)
            pltpu.sync_copy(ones, flag_sh.at[oix.at[0]], add=True)
        return carry
    lax.fori_loop(0, ROWS_O, ph1, 0)

    plsc.subcore_barrier()

    def ph2(k, carry):
        j = tid + k * NS

        @pl.when(j < R)
        def _():
            pltpu.sync_copy(w_hbm.at[pl.ds(j * 128, 128)], wv)
            pltpu.sync_copy(flag_sh.at[pl.ds(j * 128, 128)], fv)
            pltpu.sync_copy(col2d.at[pl.ds(j, 1), :], colb)
            for m in range(128 // L):
                sl = pl.ds(m * L, L)
                wvec = wv[sl]
                blended = jnp.where(fv[sl] > 0.0,
                                    (1.0 - LAMDA) * wvec + LAMDA, wvec)
                vv[sl] = jnp.exp(blended)
            pltpu.sync_copy(vv, v_sh.at[pl.ds(j * 128, 128)])
            pltpu.sync_copy(vv, seg_sh.at[colb.at[0]], add=True)
        return carry
    lax.fori_loop(0, ROWS_C, ph2, 0)

    plsc.subcore_barrier()

    pltpu.sync_copy(seg_sh, segv)

    def ph3(k, carry):
        j = wid + k * NW

        @pl.when(j < R)
        def _():
            pltpu.sync_copy(v_sh.at[pl.ds(j * 128, 128)], vv)
            pltpu.sync_copy(col2d.at[pl.ds(j, 1), :], colb)
            for m in range(128 // L):
                sl = pl.ds(m * L, L)
                cv = colb[0, sl]
                denom = plsc.load_gather(segv, [cv])
                res = vv[sl] / denom
                ov[sl] = jnp.where(res > THRESH, res, 0.0)
            pltpu.sync_copy(ov, outw.at[pl.ds(j * 128, 128)])
        return carry
    lax.fori_loop(0, ROWS_A, ph3, 0)


_kc = pl.kernel(
    _kc_body,
    out_type=jax.ShapeDtypeStruct((E,), jnp.float32),
    mesh=_mesh,
    scratch_types=(
        pltpu.VMEM_SHARED((E,), jnp.float32),
        pltpu.VMEM_SHARED((N,), jnp.float32),
        pltpu.VMEM_SHARED((E,), jnp.float32),
        pltpu.VMEM((2048,), jnp.float32),
        pltpu.VMEM((128,), jnp.float32),
        pltpu.VMEM((1, 128), jnp.int32),
        pltpu.VMEM((1, 128), jnp.int32),
        pltpu.VMEM((128,), jnp.float32),
        pltpu.VMEM((128,), jnp.float32),
        pltpu.VMEM((128,), jnp.float32),
        pltpu.VMEM((128,), jnp.float32),
        pltpu.VMEM((128,), jnp.float32),
        pltpu.VMEM((N,), jnp.float32),
    ),
    compiler_params=pltpu.CompilerParams(needs_layout_passes=False),
)


@jax.jit
def kernel(n_feat, edge_index, edge_type, ori_edge_ids, rel_table,
           W0, b0, bn_scale, bn_bias, bn_mean, bn_var, W1, b1):
    row2d = edge_index[0].reshape(R, 128)
    col2d = edge_index[1].reshape(R, 128)
    et2d = edge_type.reshape(R, 128)
    ori2d = ori_edge_ids.reshape(RO, 128)

    x1, x2f = _ka(n_feat, row2d, col2d, et2d, rel_table.reshape(-1))
    x2 = x2f.reshape(E, R_DIM)

    sc = bn_scale * lax.rsqrt(bn_var + 1e-5)
    W0s = W0 * sc[None, :]
    biasf = ((b0 - bn_mean) * sc + bn_bias).reshape(1, HID)
    logits = _kb(x1, x2, W0s[:D], W0s[D:], biasf, W1, b1.reshape(1, 1))

    out = _kc(logits.reshape(E), col2d, ori2d)
    return out.reshape(E, 1)


# speedup vs baseline: 8.4611x; 1.3543x over previous
import functools
import jax
import jax.numpy as jnp
from jax import lax
from jax.experimental import pallas as pl
from jax.experimental.pallas import tpu as pltpu
from jax.experimental.pallas import tpu_sc as plsc

N = 10000
E = 320000
E_ORI = 160000
D = 128
R_DIM = 16
N_REL = 16
HID = 64
LAMDA = 0.5
THRESH = 0.01

NC = 2
NS = 16
NW = NC * NS
L = 16

R = E // 128
RO = E_ORI // 128
ROWS_A = -(-R // NW)
ROWS_C = -(-R // NS)
ROWS_O = -(-RO // NS)

_mesh = plsc.VectorSubcoreMesh(
    core_axis_name="c", subcore_axis_name="s", num_cores=NC, num_subcores=NS)


PIPE = 78


def _ka_body(n_feat, row2d, col2d, et2d, rel_flat,
             x1, x2f,
             ridx0, ridx1, cidx0, cidx1, eidx0, eidx1,
             src0, src1, dst0, dst1, relrows0, relrows1, rel_vmem,
             semi, semg, semw):
    ridx = (ridx0, ridx1)
    cidx = (cidx0, cidx1)
    eidx = (eidx0, eidx1)
    src = (src0, src1)
    dst = (dst0, dst1)
    relrows = (relrows0, relrows1)
    c = lax.axis_index("c")
    s = lax.axis_index("s")
    wid = s * NC + c
    iota16 = lax.iota(jnp.int32, L)

    pltpu.async_copy(rel_flat, rel_vmem, semi.at[0]).wait()

    def rof(t):
        return wid + t * NW

    def issue_idx(b, t):
        r = rof(t)
        pltpu.async_copy(row2d.at[r], ridx[b], semi.at[b])
        pltpu.async_copy(col2d.at[r], cidx[b], semi.at[b])
        pltpu.async_copy(et2d.at[r], eidx[b], semi.at[b])

    def wait_idx(b, t):
        r = rof(t)
        pltpu.make_async_copy(row2d.at[r], ridx[b], semi.at[b]).wait()
        pltpu.make_async_copy(col2d.at[r], cidx[b], semi.at[b]).wait()
        pltpu.make_async_copy(et2d.at[r], eidx[b], semi.at[b]).wait()

    def issue_gather(b):
        pltpu.async_copy(n_feat.at[ridx[b]], src[b], semg.at[b])
        pltpu.async_copy(n_feat.at[cidx[b]], dst[b], semg.at[b])

    def wait_gather(b):
        pltpu.make_async_copy(n_feat.at[ridx[b]], src[b],
                              semg.at[b]).wait()
        pltpu.make_async_copy(n_feat.at[cidx[b]], dst[b],
                              semg.at[b]).wait()

    def issue_wb(b, t):
        r = rof(t)
        pltpu.async_copy(src[b], x1.at[pl.ds(r * 128, 128), :], semw.at[b])
        pltpu.async_copy(relrows[b],
                         x2f.at[pl.ds(r * 128 * R_DIM, 128 * R_DIM)],
                         semw.at[b])

    def wait_wb(b, t):
        r = rof(t)
        pltpu.make_async_copy(src[b], x1.at[pl.ds(r * 128, 128), :],
                              semw.at[b]).wait()
        pltpu.make_async_copy(relrows[b],
                              x2f.at[pl.ds(r * 128 * R_DIM, 128 * R_DIM)],
                              semw.at[b]).wait()

    def compute(b):
        def row_i(i, carry2):
            for j in range(D // L):
                sl = pl.ds(j * L, L)
                src[b][i, sl] = -jnp.abs(src[b][i, sl] - dst[b][i, sl])
            return carry2
        lax.fori_loop(0, 128, row_i, 0)

        def rel_m(m, carry2):
            ev = eidx[b][pl.ds(m * L, L)]
            base = ev * R_DIM
            dst_base = m * (L * R_DIM) + iota16 * R_DIM
            for j in range(R_DIM):
                vals = plsc.load_gather(rel_vmem, [base + j])
                plsc.store_scatter(relrows[b], [dst_base + j], vals)
            return carry2
        lax.fori_loop(0, 128 // L, rel_m, 0)

    def step(t, b, nb):
        @pl.when(t + 1 < PIPE)
        def _():
            issue_idx(nb, t + 1)
        wait_gather(b)

        @pl.when(t + 1 < PIPE)
        def _():
            wait_idx(nb, t + 1)

            @pl.when(t >= 1)
            def _():
                wait_wb(nb, t - 1)
            issue_gather(nb)
        compute(b)
        issue_wb(b, t)

    issue_idx(0, 0)
    wait_idx(0, 0)
    issue_gather(0)

    def outer(to, carry):
        step(2 * to, 0, 1)
        step(2 * to + 1, 1, 0)
        return carry
    lax.fori_loop(0, PIPE // 2, outer, 0)

    wait_wb(0, PIPE - 2)
    wait_wb(1, PIPE - 1)

    r_tail = rof(PIPE)

    @pl.when(r_tail < R)
    def _():
        issue_idx(0, PIPE)
        wait_idx(0, PIPE)
        issue_gather(0)
        wait_gather(0)
        compute(0)
        issue_wb(0, PIPE)
        wait_wb(0, PIPE)


_ka = pl.kernel(
    _ka_body,
    out_type=(
        jax.ShapeDtypeStruct((E, D), jnp.float32),
        jax.ShapeDtypeStruct((E * R_DIM,), jnp.float32),
    ),
    mesh=_mesh,
    scratch_types=(
        pltpu.VMEM((128,), jnp.int32),
        pltpu.VMEM((128,), jnp.int32),
        pltpu.VMEM((128,), jnp.int32),
        pltpu.VMEM((128,), jnp.int32),
        pltpu.VMEM((128,), jnp.int32),
        pltpu.VMEM((128,), jnp.int32),
        pltpu.VMEM((128, D), jnp.float32),
        pltpu.VMEM((128, D), jnp.float32),
        pltpu.VMEM((128, D), jnp.float32),
        pltpu.VMEM((128, D), jnp.float32),
        pltpu.VMEM((128 * R_DIM,), jnp.float32),
        pltpu.VMEM((128 * R_DIM,), jnp.float32),
        pltpu.VMEM((N_REL * R_DIM,), jnp.float32),
        pltpu.SemaphoreType.DMA((2,)),
        pltpu.SemaphoreType.DMA((2,)),
        pltpu.SemaphoreType.DMA((2,)),
    ),
    compiler_params=pltpu.CompilerParams(needs_layout_passes=False),
)


BB = 2560


def _kb_body(x1_ref, x2_ref, w0a_ref, w0b_ref, bias_ref, w1_ref, b1_ref,
             out_ref):
    e1 = jnp.exp(x1_ref[...])
    h = (jnp.dot(e1, w0a_ref[...], preferred_element_type=jnp.float32)
         + jnp.dot(x2_ref[...], w0b_ref[...],
                   preferred_element_type=jnp.float32)
         + bias_ref[...])
    h = jnp.where(h >= 0.0, h, 0.01 * h)
    w = jnp.dot(h, w1_ref[...], preferred_element_type=jnp.float32)
    out_ref[...] = w + b1_ref[0, 0]


_kb = pl.pallas_call(
    _kb_body,
    out_shape=jax.ShapeDtypeStruct((E, 1), jnp.float32),
    grid=(E // BB,),
    in_specs=[
        pl.BlockSpec((BB, D), lambda i: (i, 0)),
        pl.BlockSpec((BB, R_DIM), lambda i: (i, 0)),
        pl.BlockSpec((D, HID), lambda i: (0, 0)),
        pl.BlockSpec((R_DIM, HID), lambda i: (0, 0)),
        pl.BlockSpec((1, HID), lambda i: (0, 0)),
        pl.BlockSpec((HID, 1), lambda i: (0, 0)),
        pl.BlockSpec((1, 1), lambda i: (0, 0)),
    ],
    out_specs=pl.BlockSpec((BB, 1), lambda i: (i, 0)),
)


def _kc_body(w_hbm, col2d, ori2d,
             outw,
             flag_sh, seg_sh, v_sh,
             zb, ones, oix, colb, wv, fv, vv, sv, ov, segv):
    c = lax.axis_index("c")
    s = lax.axis_index("s")
    tid = s
    wid = s * NC + c
    zero16 = jnp.zeros((L,), jnp.float32)
    one16 = jnp.ones((L,), jnp.float32)

    def z_i(i, carry):
        zb[pl.ds(i * L, L)] = zero16
        return carry
    lax.fori_loop(0, 2048 // L, z_i, 0)

    def o_i(i, carry):
        ones[pl.ds(i * L, L)] = one16
        return carry
    lax.fori_loop(0, 128 // L, o_i, 0)

    span = E // NS

    def zf_k(k, carry):
        pltpu.sync_copy(zb, flag_sh.at[pl.ds(tid * span + k * 2048, 2048)])
        return carry
    lax.fori_loop(0, 9, zf_k, 0)
    pltpu.sync_copy(zb.at[pl.ds(0, 1568)],
                    flag_sh.at[pl.ds(tid * span + 9 * 2048, 1568)])

    @pl.when(tid < 10)
    def _():
        pltpu.sync_copy(zb.at[pl.ds(0, 1000)],
                        seg_sh.at[pl.ds(tid * 1000, 1000)])

    plsc.subcore_barrier()

    def ph1(k, carry):
        j = tid + k * NS

        @pl.when(j < RO)
        def _():
            pltpu.sync_copy(ori2d.at[pl.ds(j, 1), :], oix)
            pltpu.sync_copy(ones, flag_sh.at[oix.at[0]], add=True)
        return carry
    lax.fori_loop(0, ROWS_O, ph1, 0)

    plsc.subcore_barrier()

    def ph2(k, carry):
        j = tid + k * NS

        @pl.when(j < R)
        def _():
            pltpu.sync_copy(w_hbm.at[pl.ds(j * 128, 128)], wv)
            pltpu.sync_copy(flag_sh.at[pl.ds(j * 128, 128)], fv)
            pltpu.sync_copy(col2d.at[pl.ds(j, 1), :], colb)
            for m in range(128 // L):
                sl = pl.ds(m * L, L)
                wvec = wv[sl]
                blended = jnp.where(fv[sl] > 0.0,
                                    (1.0 - LAMDA) * wvec + LAMDA, wvec)
                vv[sl] = jnp.exp(blended)
            pltpu.sync_copy(vv, v_sh.at[pl.ds(j * 128, 128)])
            pltpu.sync_copy(vv, seg_sh.at[colb.at[0]], add=True)
        return carry
    lax.fori_loop(0, ROWS_C, ph2, 0)

    plsc.subcore_barrier()

    pltpu.sync_copy(seg_sh, segv)

    def ph3(k, carry):
        j = wid + k * NW

        @pl.when(j < R)
        def _():
            pltpu.sync_copy(v_sh.at[pl.ds(j * 128, 128)], vv)
            pltpu.sync_copy(col2d.at[pl.ds(j, 1), :], colb)
            for m in range(128 // L):
                sl = pl.ds(m * L, L)
                cv = colb[0, sl]
                denom = plsc.load_gather(segv, [cv])
                res = vv[sl] / denom
                ov[sl] = jnp.where(res > THRESH, res, 0.0)
            pltpu.sync_copy(ov, outw.at[pl.ds(j * 128, 128)])
        return carry
    lax.fori_loop(0, ROWS_A, ph3, 0)


_kc = pl.kernel(
    _kc_body,
    out_type=jax.ShapeDtypeStruct((E,), jnp.float32),
    mesh=_mesh,
    scratch_types=(
        pltpu.VMEM_SHARED((E,), jnp.float32),
        pltpu.VMEM_SHARED((N,), jnp.float32),
        pltpu.VMEM_SHARED((E,), jnp.float32),
        pltpu.VMEM((2048,), jnp.float32),
        pltpu.VMEM((128,), jnp.float32),
        pltpu.VMEM((1, 128), jnp.int32),
        pltpu.VMEM((1, 128), jnp.int32),
        pltpu.VMEM((128,), jnp.float32),
        pltpu.VMEM((128,), jnp.float32),
        pltpu.VMEM((128,), jnp.float32),
        pltpu.VMEM((128,), jnp.float32),
        pltpu.VMEM((128,), jnp.float32),
        pltpu.VMEM((N,), jnp.float32),
    ),
    compiler_params=pltpu.CompilerParams(needs_layout_passes=False),
)


@jax.jit
def kernel(n_feat, edge_index, edge_type, ori_edge_ids, rel_table,
           W0, b0, bn_scale, bn_bias, bn_mean, bn_var, W1, b1):
    row2d = edge_index[0].reshape(R, 128)
    col2d = edge_index[1].reshape(R, 128)
    et2d = edge_type.reshape(R, 128)
    ori2d = ori_edge_ids.reshape(RO, 128)

    x1, x2f = _ka(n_feat, row2d, col2d, et2d, rel_table.reshape(-1))
    x2 = x2f.reshape(E, R_DIM)

    sc = bn_scale * lax.rsqrt(bn_var + 1e-5)
    W0s = W0 * sc[None, :]
    biasf = ((b0 - bn_mean) * sc + bn_bias).reshape(1, HID)
    logits = _kb(x1, x2, W0s[:D], W0s[D:], biasf, W1, b1.reshape(1, 1))

    out = _kc(logits.reshape(E), col2d, ori2d)
    return out.reshape(E, 1)


# speedup vs baseline: 11.2208x; 1.3262x over previous
import functools
import jax
import jax.numpy as jnp
from jax import lax
from jax.experimental import pallas as pl
from jax.experimental.pallas import tpu as pltpu
from jax.experimental.pallas import tpu_sc as plsc

N = 10000
E = 320000
E_ORI = 160000
D = 128
R_DIM = 16
N_REL = 16
HID = 64
LAMDA = 0.5
THRESH = 0.01

NC = 2
NS = 16
NW = NC * NS
L = 16

R = E // 128
RO = E_ORI // 128
ROWS_A = -(-R // NW)
ROWS_C = -(-R // NS)
ROWS_O = -(-RO // NS)

_mesh = plsc.VectorSubcoreMesh(
    core_axis_name="c", subcore_axis_name="s", num_cores=NC, num_subcores=NS)


PIPE = 78


def _ka_body(n_feat, row2d, col2d, et2d, rel_flat,
             x1, x2f,
             ridx0, ridx1, cidx0, cidx1, eidx0, eidx1,
             src0, src1, dst0, dst1, relrows0, relrows1, rel_vmem,
             semi, semg, semw):
    ridx = (ridx0, ridx1)
    cidx = (cidx0, cidx1)
    eidx = (eidx0, eidx1)
    src = (src0, src1)
    dst = (dst0, dst1)
    relrows = (relrows0, relrows1)
    c = lax.axis_index("c")
    s = lax.axis_index("s")
    wid = s * NC + c
    iota16 = lax.iota(jnp.int32, L)

    pltpu.async_copy(rel_flat, rel_vmem, semi.at[0]).wait()

    def rof(t):
        return wid + t * NW

    def issue_idx(b, t):
        r = rof(t)
        pltpu.async_copy(row2d.at[r], ridx[b], semi.at[b])
        pltpu.async_copy(col2d.at[r], cidx[b], semi.at[b])
        pltpu.async_copy(et2d.at[r], eidx[b], semi.at[b])

    def wait_idx(b, t):
        r = rof(t)
        pltpu.make_async_copy(row2d.at[r], ridx[b], semi.at[b]).wait()
        pltpu.make_async_copy(col2d.at[r], cidx[b], semi.at[b]).wait()
        pltpu.make_async_copy(et2d.at[r], eidx[b], semi.at[b]).wait()

    def issue_gather(b):
        pltpu.async_copy(n_feat.at[ridx[b]], src[b], semg.at[b])
        pltpu.async_copy(n_feat.at[cidx[b]], dst[b], semg.at[b])

    def wait_gather(b):
        pltpu.make_async_copy(n_feat.at[ridx[b]], src[b],
                              semg.at[b]).wait()
        pltpu.make_async_copy(n_feat.at[cidx[b]], dst[b],
                              semg.at[b]).wait()

    def issue_wb(b, t):
        r = rof(t)
        pltpu.async_copy(src[b], x1.at[pl.ds(r * 128, 128), :], semw.at[b])
        pltpu.async_copy(relrows[b],
                         x2f.at[pl.ds(r * 128 * R_DIM, 128 * R_DIM)],
                         semw.at[b])

    def wait_wb(b, t):
        r = rof(t)
        pltpu.make_async_copy(src[b], x1.at[pl.ds(r * 128, 128), :],
                              semw.at[b]).wait()
        pltpu.make_async_copy(relrows[b],
                              x2f.at[pl.ds(r * 128 * R_DIM, 128 * R_DIM)],
                              semw.at[b]).wait()

    def compute(b):
        def row_i(i, carry2):
            for j in range(D // L):
                sl = pl.ds(j * L, L)
                src[b][i, sl] = -jnp.abs(src[b][i, sl] - dst[b][i, sl])
            return carry2
        lax.fori_loop(0, 128, row_i, 0)

        def rel_m(m, carry2):
            ev = eidx[b][pl.ds(m * L, L)]
            base = ev * R_DIM
            dst_base = m * (L * R_DIM) + iota16 * R_DIM
            for j in range(R_DIM):
                vals = plsc.load_gather(rel_vmem, [base + j])
                plsc.store_scatter(relrows[b], [dst_base + j], vals)
            return carry2
        lax.fori_loop(0, 128 // L, rel_m, 0)

    def step(t, b, nb):
        @pl.when(t + 1 < PIPE)
        def _():
            issue_idx(nb, t + 1)
        wait_gather(b)

        @pl.when(t + 1 < PIPE)
        def _():
            wait_idx(nb, t + 1)

            @pl.when(t >= 1)
            def _():
                wait_wb(nb, t - 1)
            issue_gather(nb)
        compute(b)
        issue_wb(b, t)

    issue_idx(0, 0)
    wait_idx(0, 0)
    issue_gather(0)

    def outer(to, carry):
        step(2 * to, 0, 1)
        step(2 * to + 1, 1, 0)
        return carry
    lax.fori_loop(0, PIPE // 2, outer, 0)

    wait_wb(0, PIPE - 2)
    wait_wb(1, PIPE - 1)

    r_tail = rof(PIPE)

    @pl.when(r_tail < R)
    def _():
        issue_idx(0, PIPE)
        wait_idx(0, PIPE)
        issue_gather(0)
        wait_gather(0)
        compute(0)
        issue_wb(0, PIPE)
        wait_wb(0, PIPE)


_ka = pl.kernel(
    _ka_body,
    out_type=(
        jax.ShapeDtypeStruct((E, D), jnp.float32),
        jax.ShapeDtypeStruct((E * R_DIM,), jnp.float32),
    ),
    mesh=_mesh,
    scratch_types=(
        pltpu.VMEM((128,), jnp.int32),
        pltpu.VMEM((128,), jnp.int32),
        pltpu.VMEM((128,), jnp.int32),
        pltpu.VMEM((128,), jnp.int32),
        pltpu.VMEM((128,), jnp.int32),
        pltpu.VMEM((128,), jnp.int32),
        pltpu.VMEM((128, D), jnp.float32),
        pltpu.VMEM((128, D), jnp.float32),
        pltpu.VMEM((128, D), jnp.float32),
        pltpu.VMEM((128, D), jnp.float32),
        pltpu.VMEM((128 * R_DIM,), jnp.float32),
        pltpu.VMEM((128 * R_DIM,), jnp.float32),
        pltpu.VMEM((N_REL * R_DIM,), jnp.float32),
        pltpu.SemaphoreType.DMA((2,)),
        pltpu.SemaphoreType.DMA((2,)),
        pltpu.SemaphoreType.DMA((2,)),
    ),
    compiler_params=pltpu.CompilerParams(needs_layout_passes=False),
)


BB = 2560


def _kb_body(x1_ref, x2_ref, w0a_ref, w0b_ref, bias_ref, w1_ref, b1_ref,
             out_ref):
    e1 = jnp.exp(x1_ref[...])
    h = (jnp.dot(e1, w0a_ref[...], preferred_element_type=jnp.float32)
         + jnp.dot(x2_ref[...], w0b_ref[...],
                   preferred_element_type=jnp.float32)
         + bias_ref[...])
    h = jnp.where(h >= 0.0, h, 0.01 * h)
    w = jnp.dot(h, w1_ref[...], preferred_element_type=jnp.float32)
    out_ref[...] = w + b1_ref[0, 0]


_kb = pl.pallas_call(
    _kb_body,
    out_shape=jax.ShapeDtypeStruct((E, 1), jnp.float32),
    grid=(E // BB,),
    in_specs=[
        pl.BlockSpec((BB, D), lambda i: (i, 0)),
        pl.BlockSpec((BB, R_DIM), lambda i: (i, 0)),
        pl.BlockSpec((D, HID), lambda i: (0, 0)),
        pl.BlockSpec((R_DIM, HID), lambda i: (0, 0)),
        pl.BlockSpec((1, HID), lambda i: (0, 0)),
        pl.BlockSpec((HID, 1), lambda i: (0, 0)),
        pl.BlockSpec((1, 1), lambda i: (0, 0)),
    ],
    out_specs=pl.BlockSpec((BB, 1), lambda i: (i, 0)),
)


RB = 8
T2 = 160
T1 = 80
T3 = 80


def _kc_body(w_hbm, col2d, ori2d,
             outw,
             flag_sh, seg_sh, v_sh,
             zb, ones, oix8, colb8, wv8, fv8, vv8, ov8, segv, semc):
    c = lax.axis_index("c")
    s = lax.axis_index("s")
    tid = s
    wid = s * NC + c
    zero16 = jnp.zeros((L,), jnp.float32)
    one16 = jnp.ones((L,), jnp.float32)

    def z_i(i, carry):
        zb[pl.ds(i * L, L)] = zero16
        return carry
    lax.fori_loop(0, 2048 // L, z_i, 0)

    def o_i(i, carry):
        ones[pl.ds(i * L, L)] = one16
        return carry
    lax.fori_loop(0, 128 // L, o_i, 0)

    span = E // NS

    def zf_k(k, carry):
        pltpu.sync_copy(zb, flag_sh.at[pl.ds(tid * span + k * 2048, 2048)])
        return carry
    lax.fori_loop(0, 9, zf_k, 0)
    pltpu.sync_copy(zb.at[pl.ds(0, 1568)],
                    flag_sh.at[pl.ds(tid * span + 9 * 2048, 1568)])

    @pl.when(tid < 10)
    def _():
        pltpu.sync_copy(zb.at[pl.ds(0, 1000)],
                        seg_sh.at[pl.ds(tid * 1000, 1000)])

    plsc.subcore_barrier()

    def ph1_batch(jb):
        pltpu.sync_copy(ori2d.at[pl.ds(jb, RB), :], oix8)
        descs = []
        for q in range(RB):
            descs.append(pltpu.async_copy(
                ones, flag_sh.at[oix8.at[q]], semc, add=True))
        for d in descs:
            d.wait()

    def ph1_row(j):
        pltpu.sync_copy(ori2d.at[pl.ds(j, 1), :],
                        oix8.at[pl.ds(0, 1), :])
        pltpu.sync_copy(ones, flag_sh.at[oix8.at[0]], add=True)

    def ph1(kb, carry):
        jb = tid * T1 + kb * RB

        @pl.when(jb + RB <= RO)
        def _():
            ph1_batch(jb)

        @pl.when(jb + RB > RO)
        def _():
            for i in range(RB):
                @pl.when(jb + i < RO)
                def _():
                    ph1_row(jb + i)
        return carry
    lax.fori_loop(0, T1 // RB, ph1, 0)

    plsc.subcore_barrier()

    def ph2_body(jb, nrows):
        ne = nrows * 128
        pltpu.sync_copy(w_hbm.at[pl.ds(jb * 128, ne)],
                        wv8.at[pl.ds(0, ne)])
        pltpu.sync_copy(flag_sh.at[pl.ds(jb * 128, ne)],
                        fv8.at[pl.ds(0, ne)])
        pltpu.sync_copy(col2d.at[pl.ds(jb, nrows), :],
                        colb8.at[pl.ds(0, nrows), :])
        for m in range(ne // L):
            sl = pl.ds(m * L, L)
            wvec = wv8[sl]
            blended = jnp.where(fv8[sl] > 0.0,
                                (1.0 - LAMDA) * wvec + LAMDA, wvec)
            vv8[sl] = jnp.exp(blended)
        pltpu.sync_copy(vv8.at[pl.ds(0, ne)], v_sh.at[pl.ds(jb * 128, ne)])
        descs = []
        for q in range(nrows):
            descs.append(pltpu.async_copy(
                vv8.at[pl.ds(q * 128, 128)], seg_sh.at[colb8.at[q]],
                semc, add=True))
        for d in descs:
            d.wait()

    def ph2(kb, carry):
        jb = tid * T2 + kb * RB

        @pl.when(jb + RB <= R)
        def _():
            ph2_body(jb, RB)

        @pl.when(jb + RB > R)
        def _():
            for i in range(RB):
                @pl.when(jb + i < R)
                def _():
                    ph2_body(jb + i, 1)
        return carry
    lax.fori_loop(0, T2 // RB, ph2, 0)

    plsc.subcore_barrier()

    pltpu.sync_copy(seg_sh, segv)

    def ph3_body(jb, nrows):
        ne = nrows * 128
        pltpu.sync_copy(v_sh.at[pl.ds(jb * 128, ne)],
                        vv8.at[pl.ds(0, ne)])
        pltpu.sync_copy(col2d.at[pl.ds(jb, nrows), :],
                        colb8.at[pl.ds(0, nrows), :])
        for q in range(nrows):
            for m in range(128 // L):
                sl = pl.ds(q * 128 + m * L, L)
                cv = colb8[q, pl.ds(m * L, L)]
                denom = plsc.load_gather(segv, [cv])
                res = vv8[sl] / denom
                ov8[sl] = jnp.where(res > THRESH, res, 0.0)
        pltpu.sync_copy(ov8.at[pl.ds(0, ne)], outw.at[pl.ds(jb * 128, ne)])

    def ph3(kb, carry):
        jb = wid * T3 + kb * RB

        @pl.when(jb + RB <= R)
        def _():
            ph3_body(jb, RB)

        @pl.when(jb + RB > R)
        def _():
            for i in range(RB):
                @pl.when(jb + i < R)
                def _():
                    ph3_body(jb + i, 1)
        return carry
    lax.fori_loop(0, T3 // RB, ph3, 0)


_kc = pl.kernel(
    _kc_body,
    out_type=jax.ShapeDtypeStruct((E,), jnp.float32),
    mesh=_mesh,
    scratch_types=(
        pltpu.VMEM_SHARED((E,), jnp.float32),
        pltpu.VMEM_SHARED((N,), jnp.float32),
        pltpu.VMEM_SHARED((E,), jnp.float32),
        pltpu.VMEM((2048,), jnp.float32),
        pltpu.VMEM((128,), jnp.float32),
        pltpu.VMEM((RB, 128), jnp.int32),
        pltpu.VMEM((RB, 128), jnp.int32),
        pltpu.VMEM((RB * 128,), jnp.float32),
        pltpu.VMEM((RB * 128,), jnp.float32),
        pltpu.VMEM((RB * 128,), jnp.float32),
        pltpu.VMEM((RB * 128,), jnp.float32),
        pltpu.VMEM((N,), jnp.float32),
        pltpu.SemaphoreType.DMA,
    ),
    compiler_params=pltpu.CompilerParams(needs_layout_passes=False),
)


@jax.jit
def kernel(n_feat, edge_index, edge_type, ori_edge_ids, rel_table,
           W0, b0, bn_scale, bn_bias, bn_mean, bn_var, W1, b1):
    row2d = edge_index[0].reshape(R, 128)
    col2d = edge_index[1].reshape(R, 128)
    et2d = edge_type.reshape(R, 128)
    ori2d = ori_edge_ids.reshape(RO, 128)

    x1, x2f = _ka(n_feat, row2d, col2d, et2d, rel_table.reshape(-1))
    x2 = x2f.reshape(E, R_DIM)

    sc = bn_scale * lax.rsqrt(bn_var + 1e-5)
    W0s = W0 * sc[None, :]
    biasf = ((b0 - bn_mean) * sc + bn_bias).reshape(1, HID)
    logits = _kb(x1, x2, W0s[:D], W0s[D:], biasf, W1, b1.reshape(1, 1))

    out = _kc(logits.reshape(E), col2d, ori2d)
    return out.reshape(E, 1)


# speedup vs baseline: 11.4377x; 1.0193x over previous
import functools
import jax
import jax.numpy as jnp
from jax import lax
from jax.experimental import pallas as pl
from jax.experimental.pallas import tpu as pltpu
from jax.experimental.pallas import tpu_sc as plsc

N = 10000
E = 320000
E_ORI = 160000
D = 128
R_DIM = 16
N_REL = 16
HID = 64
LAMDA = 0.5
THRESH = 0.01

NC = 2
NS = 16
NW = NC * NS
L = 16

R = E // 128
RO = E_ORI // 128
ROWS_A = -(-R // NW)
ROWS_C = -(-R // NS)
ROWS_O = -(-RO // NS)

_mesh = plsc.VectorSubcoreMesh(
    core_axis_name="c", subcore_axis_name="s", num_cores=NC, num_subcores=NS)


PIPE = 78


def _ka_body(n_feat, row2d, col2d, et2d, rel_flat,
             x1, x2f,
             ridx0, ridx1, cidx0, cidx1, eidx0, eidx1,
             src0, src1, dst0, dst1, relrows0, relrows1, rel_vmem,
             semi, semg, semw):
    ridx = (ridx0, ridx1)
    cidx = (cidx0, cidx1)
    eidx = (eidx0, eidx1)
    src = (src0, src1)
    dst = (dst0, dst1)
    relrows = (relrows0, relrows1)
    c = lax.axis_index("c")
    s = lax.axis_index("s")
    wid = s * NC + c
    iota16 = lax.iota(jnp.int32, L)

    pltpu.async_copy(rel_flat, rel_vmem, semi.at[0]).wait()

    def rof(t):
        return wid + t * NW

    def issue_idx(b, t):
        r = rof(t)
        pltpu.async_copy(row2d.at[r], ridx[b], semi.at[b])
        pltpu.async_copy(col2d.at[r], cidx[b], semi.at[b])
        pltpu.async_copy(et2d.at[r], eidx[b], semi.at[b])

    def wait_idx(b, t):
        r = rof(t)
        pltpu.make_async_copy(row2d.at[r], ridx[b], semi.at[b]).wait()
        pltpu.make_async_copy(col2d.at[r], cidx[b], semi.at[b]).wait()
        pltpu.make_async_copy(et2d.at[r], eidx[b], semi.at[b]).wait()

    def issue_gather(b):
        pltpu.async_copy(n_feat.at[ridx[b]], src[b], semg.at[b])
        pltpu.async_copy(n_feat.at[cidx[b]], dst[b], semg.at[b])

    def wait_gather(b):
        pltpu.make_async_copy(n_feat.at[ridx[b]], src[b],
                              semg.at[b]).wait()
        pltpu.make_async_copy(n_feat.at[cidx[b]], dst[b],
                              semg.at[b]).wait()

    def issue_wb(b, t):
        r = rof(t)
        pltpu.async_copy(src[b], x1.at[pl.ds(r * 128, 128), :], semw.at[b])
        pltpu.async_copy(relrows[b],
                         x2f.at[pl.ds(r * 128 * R_DIM, 128 * R_DIM)],
                         semw.at[b])

    def wait_wb(b, t):
        r = rof(t)
        pltpu.make_async_copy(src[b], x1.at[pl.ds(r * 128, 128), :],
                              semw.at[b]).wait()
        pltpu.make_async_copy(relrows[b],
                              x2f.at[pl.ds(r * 128 * R_DIM, 128 * R_DIM)],
                              semw.at[b]).wait()

    def compute(b):
        def row_i(i, carry2):
            for j in range(D // L):
                sl = pl.ds(j * L, L)
                src[b][i, sl] = -jnp.abs(src[b][i, sl] - dst[b][i, sl])
            return carry2
        lax.fori_loop(0, 128, row_i, 0)

        def rel_m(m, carry2):
            ev = eidx[b][pl.ds(m * L, L)]
            base = ev * R_DIM
            dst_base = m * (L * R_DIM) + iota16 * R_DIM
            for j in range(R_DIM):
                vals = plsc.load_gather(rel_vmem, [base + j])
                plsc.store_scatter(relrows[b], [dst_base + j], vals)
            return carry2
        lax.fori_loop(0, 128 // L, rel_m, 0)

    def step(t, b, nb):
        @pl.when(t + 1 < PIPE)
        def _():
            issue_idx(nb, t + 1)
        wait_gather(b)

        @pl.when(t + 1 < PIPE)
        def _():
            wait_idx(nb, t + 1)

            @pl.when(t >= 1)
            def _():
                wait_wb(nb, t - 1)
            issue_gather(nb)
        compute(b)
        issue_wb(b, t)

    issue_idx(0, 0)
    wait_idx(0, 0)
    issue_gather(0)

    def outer(to, carry):
        step(2 * to, 0, 1)
        step(2 * to + 1, 1, 0)
        return carry
    lax.fori_loop(0, PIPE // 2, outer, 0)

    wait_wb(0, PIPE - 2)
    wait_wb(1, PIPE - 1)

    r_tail = rof(PIPE)

    @pl.when(r_tail < R)
    def _():
        issue_idx(0, PIPE)
        wait_idx(0, PIPE)
        issue_gather(0)
        wait_gather(0)
        compute(0)
        issue_wb(0, PIPE)
        wait_wb(0, PIPE)


_ka = pl.kernel(
    _ka_body,
    out_type=(
        jax.ShapeDtypeStruct((E, D), jnp.float32),
        jax.ShapeDtypeStruct((E * R_DIM,), jnp.float32),
    ),
    mesh=_mesh,
    scratch_types=(
        pltpu.VMEM((128,), jnp.int32),
        pltpu.VMEM((128,), jnp.int32),
        pltpu.VMEM((128,), jnp.int32),
        pltpu.VMEM((128,), jnp.int32),
        pltpu.VMEM((128,), jnp.int32),
        pltpu.VMEM((128,), jnp.int32),
        pltpu.VMEM((128, D), jnp.float32),
        pltpu.VMEM((128, D), jnp.float32),
        pltpu.VMEM((128, D), jnp.float32),
        pltpu.VMEM((128, D), jnp.float32),
        pltpu.VMEM((128 * R_DIM,), jnp.float32),
        pltpu.VMEM((128 * R_DIM,), jnp.float32),
        pltpu.VMEM((N_REL * R_DIM,), jnp.float32),
        pltpu.SemaphoreType.DMA((2,)),
        pltpu.SemaphoreType.DMA((2,)),
        pltpu.SemaphoreType.DMA((2,)),
    ),
    compiler_params=pltpu.CompilerParams(needs_layout_passes=False),
)


BB = 2560


def _kb_body(x1_ref, x2_ref, w0a_ref, w0b_ref, bias_ref, w1_ref, b1_ref,
             out_ref):
    e1 = jnp.exp(x1_ref[...])
    h = (jnp.dot(e1, w0a_ref[...], preferred_element_type=jnp.float32)
         + jnp.dot(x2_ref[...], w0b_ref[...],
                   preferred_element_type=jnp.float32)
         + bias_ref[...])
    h = jnp.where(h >= 0.0, h, 0.01 * h)
    w = jnp.dot(h, w1_ref[...], preferred_element_type=jnp.float32)
    out_ref[...] = w + b1_ref[0, 0]


_kb = pl.pallas_call(
    _kb_body,
    out_shape=jax.ShapeDtypeStruct((E, 1), jnp.float32),
    grid=(E // BB,),
    in_specs=[
        pl.BlockSpec((BB, D), lambda i: (i, 0)),
        pl.BlockSpec((BB, R_DIM), lambda i: (i, 0)),
        pl.BlockSpec((D, HID), lambda i: (0, 0)),
        pl.BlockSpec((R_DIM, HID), lambda i: (0, 0)),
        pl.BlockSpec((1, HID), lambda i: (0, 0)),
        pl.BlockSpec((HID, 1), lambda i: (0, 0)),
        pl.BlockSpec((1, 1), lambda i: (0, 0)),
    ],
    out_specs=pl.BlockSpec((BB, 1), lambda i: (i, 0)),
)


RB = 16
T2 = 160
T1 = 80
T3 = 80


def _kc_body(w_hbm, col2d, ori2d,
             outw,
             flag_sh, seg_sh, v_sh,
             zb, ones, oix8, colb8, wv8, fv8, vv8, ov8, segv, semc):
    c = lax.axis_index("c")
    s = lax.axis_index("s")
    tid = s
    wid = s * NC + c
    zero16 = jnp.zeros((L,), jnp.float32)
    one16 = jnp.ones((L,), jnp.float32)

    def z_i(i, carry):
        zb[pl.ds(i * L, L)] = zero16
        return carry
    lax.fori_loop(0, 2048 // L, z_i, 0)

    def o_i(i, carry):
        ones[pl.ds(i * L, L)] = one16
        return carry
    lax.fori_loop(0, 128 // L, o_i, 0)

    span = E // NS

    def zf_k(k, carry):
        pltpu.sync_copy(zb, flag_sh.at[pl.ds(tid * span + k * 2048, 2048)])
        return carry
    lax.fori_loop(0, 9, zf_k, 0)
    pltpu.sync_copy(zb.at[pl.ds(0, 1568)],
                    flag_sh.at[pl.ds(tid * span + 9 * 2048, 1568)])

    @pl.when(tid < 10)
    def _():
        pltpu.sync_copy(zb.at[pl.ds(0, 1000)],
                        seg_sh.at[pl.ds(tid * 1000, 1000)])

    plsc.subcore_barrier()

    def ph1_batch(jb):
        pltpu.sync_copy(ori2d.at[pl.ds(jb, RB), :], oix8)
        descs = []
        for q in range(RB):
            descs.append(pltpu.async_copy(
                ones, flag_sh.at[oix8.at[q]], semc, add=True))
        for d in descs:
            d.wait()

    def ph1_row(j):
        pltpu.sync_copy(ori2d.at[pl.ds(j, 1), :],
                        oix8.at[pl.ds(0, 1), :])
        pltpu.sync_copy(ones, flag_sh.at[oix8.at[0]], add=True)

    def ph1(kb, carry):
        jb = tid * T1 + kb * RB

        @pl.when(jb + RB <= RO)
        def _():
            ph1_batch(jb)

        @pl.when(jb + RB > RO)
        def _():
            for i in range(RB):
                @pl.when(jb + i < RO)
                def _():
                    ph1_row(jb + i)
        return carry
    lax.fori_loop(0, T1 // RB, ph1, 0)

    plsc.subcore_barrier()

    def ph2_body(jb, nrows):
        ne = nrows * 128
        pltpu.sync_copy(w_hbm.at[pl.ds(jb * 128, ne)],
                        wv8.at[pl.ds(0, ne)])
        pltpu.sync_copy(flag_sh.at[pl.ds(jb * 128, ne)],
                        fv8.at[pl.ds(0, ne)])
        pltpu.sync_copy(col2d.at[pl.ds(jb, nrows), :],
                        colb8.at[pl.ds(0, nrows), :])
        for m in range(ne // L):
            sl = pl.ds(m * L, L)
            wvec = wv8[sl]
            blended = jnp.where(fv8[sl] > 0.0,
                                (1.0 - LAMDA) * wvec + LAMDA, wvec)
            vv8[sl] = jnp.exp(blended)
        pltpu.sync_copy(vv8.at[pl.ds(0, ne)], v_sh.at[pl.ds(jb * 128, ne)])
        descs = []
        for q in range(nrows):
            descs.append(pltpu.async_copy(
                vv8.at[pl.ds(q * 128, 128)], seg_sh.at[colb8.at[q]],
                semc, add=True))
        for d in descs:
            d.wait()

    def ph2(kb, carry):
        jb = tid * T2 + kb * RB

        @pl.when(jb + RB <= R)
        def _():
            ph2_body(jb, RB)

        @pl.when(jb + RB > R)
        def _():
            for i in range(RB):
                @pl.when(jb + i < R)
                def _():
                    ph2_body(jb + i, 1)
        return carry
    lax.fori_loop(0, T2 // RB, ph2, 0)

    plsc.subcore_barrier()

    pltpu.sync_copy(seg_sh, segv)

    def ph3_body(jb, nrows):
        ne = nrows * 128
        pltpu.sync_copy(v_sh.at[pl.ds(jb * 128, ne)],
                        vv8.at[pl.ds(0, ne)])
        pltpu.sync_copy(col2d.at[pl.ds(jb, nrows), :],
                        colb8.at[pl.ds(0, nrows), :])
        for q in range(nrows):
            for m in range(128 // L):
                sl = pl.ds(q * 128 + m * L, L)
                cv = colb8[q, pl.ds(m * L, L)]
                denom = plsc.load_gather(segv, [cv])
                res = vv8[sl] / denom
                ov8[sl] = jnp.where(res > THRESH, res, 0.0)
        pltpu.sync_copy(ov8.at[pl.ds(0, ne)], outw.at[pl.ds(jb * 128, ne)])

    def ph3(kb, carry):
        jb = wid * T3 + kb * RB

        @pl.when(jb + RB <= R)
        def _():
            ph3_body(jb, RB)

        @pl.when(jb + RB > R)
        def _():
            for i in range(RB):
                @pl.when(jb + i < R)
                def _():
                    ph3_body(jb + i, 1)
        return carry
    lax.fori_loop(0, T3 // RB, ph3, 0)


_kc = pl.kernel(
    _kc_body,
    out_type=jax.ShapeDtypeStruct((E,), jnp.float32),
    mesh=_mesh,
    scratch_types=(
        pltpu.VMEM_SHARED((E,), jnp.float32),
        pltpu.VMEM_SHARED((N,), jnp.float32),
        pltpu.VMEM_SHARED((E,), jnp.float32),
        pltpu.VMEM((2048,), jnp.float32),
        pltpu.VMEM((128,), jnp.float32),
        pltpu.VMEM((RB, 128), jnp.int32),
        pltpu.VMEM((RB, 128), jnp.int32),
        pltpu.VMEM((RB * 128,), jnp.float32),
        pltpu.VMEM((RB * 128,), jnp.float32),
        pltpu.VMEM((RB * 128,), jnp.float32),
        pltpu.VMEM((RB * 128,), jnp.float32),
        pltpu.VMEM((N,), jnp.float32),
        pltpu.SemaphoreType.DMA,
    ),
    compiler_params=pltpu.CompilerParams(needs_layout_passes=False),
)


@jax.jit
def kernel(n_feat, edge_index, edge_type, ori_edge_ids, rel_table,
           W0, b0, bn_scale, bn_bias, bn_mean, bn_var, W1, b1):
    row2d = edge_index[0].reshape(R, 128)
    col2d = edge_index[1].reshape(R, 128)
    et2d = edge_type.reshape(R, 128)
    ori2d = ori_edge_ids.reshape(RO, 128)

    x1, x2f = _ka(n_feat, row2d, col2d, et2d, rel_table.reshape(-1))
    x2 = x2f.reshape(E, R_DIM)

    sc = bn_scale * lax.rsqrt(bn_var + 1e-5)
    W0s = W0 * sc[None, :]
    biasf = ((b0 - bn_mean) * sc + bn_bias).reshape(1, HID)
    logits = _kb(x1, x2, W0s[:D], W0s[D:], biasf, W1, b1.reshape(1, 1))

    out = _kc(logits.reshape(E), col2d, ori2d)
    return out.reshape(E, 1)
